# baseline (speedup 1.0000x reference)
import functools

import jax
import jax.numpy as jnp
from jax.experimental import pallas as pl

_N_DIS = 4096
_N_GENE = 8192
_D = 256
_OUT = 64
_ROW_BLK = 256


def _combine1_body(s_ref, c_ref, x_ref, wl_ref, wr_ref, b_ref, o_ref):
    inv = 1.0 / jnp.maximum(c_ref[:, 0:1], 1.0)
    mean = s_ref[...] * inv
    o_ref[...] = (
        jnp.dot(mean, wl_ref[...], preferred_element_type=jnp.float32)
        + jnp.dot(x_ref[...], wr_ref[...], preferred_element_type=jnp.float32)
        + b_ref[...]
    )


def _combine1(S, cnt, x_dst, Wl, Wr, b):
    n = S.shape[0]
    grid = (n // _ROW_BLK,)
    return pl.pallas_call(
        _combine1_body,
        grid=grid,
        in_specs=[
            pl.BlockSpec((_ROW_BLK, _D), lambda i: (i, 0)),
            pl.BlockSpec((_ROW_BLK, 1), lambda i: (i, 0)),
            pl.BlockSpec((_ROW_BLK, _D), lambda i: (i, 0)),
            pl.BlockSpec((_D, _D), lambda i: (0, 0)),
            pl.BlockSpec((_D, _D), lambda i: (0, 0)),
            pl.BlockSpec((1, _D), lambda i: (0, 0)),
        ],
        out_specs=pl.BlockSpec((_ROW_BLK, _D), lambda i: (i, 0)),
        out_shape=jax.ShapeDtypeStruct((n, _D), jnp.float32),
    )(S, cnt, x_dst, Wl, Wr, b.reshape(1, _D))


def _combine2_head_body(s_ref, c_ref, x_ref, wl_ref, wr_ref, b_ref,
                        wmu_ref, bmu_ref, o_ref):
    inv = 1.0 / jnp.maximum(c_ref[:, 0:1], 1.0)
    mean = s_ref[...] * inv
    h = (
        jnp.dot(mean, wl_ref[...], preferred_element_type=jnp.float32)
        + jnp.dot(x_ref[...], wr_ref[...], preferred_element_type=jnp.float32)
        + b_ref[...]
    )
    o_ref[...] = jnp.dot(h, wmu_ref[...], preferred_element_type=jnp.float32) + bmu_ref[...]


def _combine2_head(S, cnt, x_dst, Wl, Wr, b, Wmu, bmu):
    n = S.shape[0]
    grid = (n // _ROW_BLK,)
    return pl.pallas_call(
        _combine2_head_body,
        grid=grid,
        in_specs=[
            pl.BlockSpec((_ROW_BLK, _D), lambda i: (i, 0)),
            pl.BlockSpec((_ROW_BLK, 1), lambda i: (i, 0)),
            pl.BlockSpec((_ROW_BLK, _D), lambda i: (i, 0)),
            pl.BlockSpec((_D, _D), lambda i: (0, 0)),
            pl.BlockSpec((_D, _D), lambda i: (0, 0)),
            pl.BlockSpec((1, _D), lambda i: (0, 0)),
            pl.BlockSpec((_D, _OUT), lambda i: (0, 0)),
            pl.BlockSpec((1, _OUT), lambda i: (0, 0)),
        ],
        out_specs=pl.BlockSpec((_ROW_BLK, _OUT), lambda i: (i, 0)),
        out_shape=jax.ShapeDtypeStruct((n, _OUT), jnp.float32),
    )(S, cnt, x_dst, Wl, Wr, b.reshape(1, _D), Wmu, bmu.reshape(1, _OUT))


def _decoder_body(d_ref, g_ref, o_ref):
    o_ref[...] = jax.lax.dot_general(
        d_ref[...], g_ref[...], (((1,), (1,)), ((), ())),
        preferred_element_type=jnp.float32)


def _decoder(mu_d, mu_g):
    bm, bn = 512, 1024
    grid = (_N_DIS // bm, _N_GENE // bn)
    return pl.pallas_call(
        _decoder_body,
        grid=grid,
        in_specs=[
            pl.BlockSpec((bm, _OUT), lambda i, j: (i, 0)),
            pl.BlockSpec((bn, _OUT), lambda i, j: (j, 0)),
        ],
        out_specs=pl.BlockSpec((bm, bn), lambda i, j: (i, j)),
        out_shape=jax.ShapeDtypeStruct((_N_DIS, _N_GENE), jnp.float32),
    )(mu_d, mu_g)


def _seg_sum(x_src, src, dst, n_dst):
    msg = jnp.take(x_src, src, axis=0)
    return jax.ops.segment_sum(msg, dst, num_segments=n_dst)


def kernel(x_disease, x_gene, edge_index_d2g, edge_index_g2d,
           W_l1_d2g, W_r1_d2g, b1_d2g, W_l1_g2d, W_r1_g2d, b1_g2d,
           W_l2_d2g, W_r2_d2g, b2_d2g, W_l2_g2d, W_r2_g2d, b2_g2d,
           W_mu_d, b_mu_d, W_lv_d, b_lv_d, W_mu_g, b_mu_g, W_lv_g, b_lv_g):
    s1, d1i = edge_index_d2g[0], edge_index_d2g[1]
    s2, d2i = edge_index_g2d[0], edge_index_g2d[1]
    ones = jnp.ones((s1.shape[0],), jnp.float32)
    cnt_g = jax.ops.segment_sum(ones, d1i, num_segments=_N_GENE).reshape(_N_GENE, 1)
    cnt_d = jax.ops.segment_sum(ones, d2i, num_segments=_N_DIS).reshape(_N_DIS, 1)

    S1g = _seg_sum(x_disease, s1, d1i, _N_GENE)
    S1d = _seg_sum(x_gene, s2, d2i, _N_DIS)
    g1 = _combine1(S1g, cnt_g, x_gene, W_l1_d2g, W_r1_d2g, b1_d2g)
    d1 = _combine1(S1d, cnt_d, x_disease, W_l1_g2d, W_r1_g2d, b1_g2d)

    S2g = _seg_sum(d1, s1, d1i, _N_GENE)
    S2d = _seg_sum(g1, s2, d2i, _N_DIS)
    mu_g = _combine2_head(S2g, cnt_g, g1, W_l2_d2g, W_r2_d2g, b2_d2g, W_mu_g, b_mu_g)
    mu_d = _combine2_head(S2d, cnt_d, d1, W_l2_g2d, W_r2_g2d, b2_g2d, W_mu_d, b_mu_d)

    return _decoder(mu_d, mu_g)


# speedup vs baseline: 3.8379x; 3.8379x over previous
import functools

import jax
import jax.numpy as jnp
from jax import lax
from jax.experimental import pallas as pl
from jax.experimental.pallas import tpu as pltpu
from jax.experimental.pallas import tpu_sc as plsc

_N_DIS = 4096
_N_GENE = 8192
_E = 262144
_D = 256
_HD = 128
_OUT = 64
_ROW_BLK = 256
_C = 128
_NS = 16
_EPS = _E // _NS
_NCH = _EPS // _C


def _mesh():
    return plsc.VectorSubcoreMesh(core_axis_name="c", subcore_axis_name="s")


def _sc_layer(xA_cat, xB_cat, s1o, d1i, s2o, d2i, zer):
    gs = _N_GENE // _NS
    ds_ = _N_DIS // _NS

    @functools.partial(
        pl.kernel,
        out_type=[
            jax.ShapeDtypeStruct((2 * _N_GENE, _HD), jnp.float32),
            jax.ShapeDtypeStruct((2 * _N_DIS, _HD), jnp.float32),
        ],
        mesh=_mesh(),
        scratch_types=[
            pltpu.VMEM((_C,), jnp.int32),
            pltpu.VMEM((_C,), jnp.int32),
            pltpu.VMEM((_C, _HD), jnp.float32),
            pltpu.VMEM_SHARED((_N_GENE, _HD), jnp.float32),
            pltpu.VMEM_SHARED((_N_DIS, _HD), jnp.float32),
            pltpu.SemaphoreType.DMA,
        ],
    )
    def k(xA, xB, s1r, d1r, s2r, d2r, zr, sg, sd,
          idx_s, idx_d, gbuf, acc_g, acc_d, sem):
        c = lax.axis_index("c")
        s = lax.axis_index("s")
        pltpu.sync_copy(zr, acc_g.at[pl.ds(s * gs, gs)])
        pltpu.sync_copy(zr.at[pl.ds(0, ds_)], acc_d.at[pl.ds(s * ds_, ds_)])
        plsc.subcore_barrier()

        def make_pass(table, srcr, dstr, acc):
            def body(i, carry):
                base = s * _EPS + i * _C
                pltpu.sync_copy(srcr.at[pl.ds(c * _E + base, _C)], idx_s)
                pltpu.sync_copy(dstr.at[pl.ds(base, _C)], idx_d)
                pltpu.async_copy(table.at[idx_s], gbuf, sem).wait()
                pltpu.sync_copy(gbuf, acc.at[idx_d], add=True)
                return carry
            return body

        lax.fori_loop(0, _NCH, make_pass(xA, s1r, d1r, acc_g), 0)
        lax.fori_loop(0, _NCH, make_pass(xB, s2r, d2r, acc_d), 0)
        plsc.subcore_barrier()
        pltpu.sync_copy(acc_g.at[pl.ds(s * gs, gs)],
                        sg.at[pl.ds(c * _N_GENE + s * gs, gs)])
        pltpu.sync_copy(acc_d.at[pl.ds(s * ds_, ds_)],
                        sd.at[pl.ds(c * _N_DIS + s * ds_, ds_)])

    return k(xA_cat, xB_cat, s1o, d1i, s2o, d2i, zer)


def _sc_counts(dcat, ones_r, zer):
    gs = _N_GENE // _NS
    ds_ = _N_DIS // _NS

    @functools.partial(
        pl.kernel,
        out_type=jax.ShapeDtypeStruct((_N_GENE + _N_DIS, _HD), jnp.float32),
        mesh=_mesh(),
        scratch_types=[
            pltpu.VMEM((_C,), jnp.int32),
            pltpu.VMEM((_C, _HD), jnp.float32),
            pltpu.VMEM_SHARED((_N_GENE, _HD), jnp.float32),
        ],
    )
    def k(dr, onesr, zr, cnt, idx_d, obuf, acc):
        c = lax.axis_index("c")
        s = lax.axis_index("s")
        pltpu.sync_copy(zr, acc.at[pl.ds(s * gs, gs)])
        pltpu.sync_copy(onesr, obuf)
        plsc.subcore_barrier()

        def body(i, carry):
            base = s * _EPS + i * _C
            pltpu.sync_copy(dr.at[pl.ds(c * _E + base, _C)], idx_d)
            pltpu.sync_copy(obuf, acc.at[idx_d], add=True)
            return carry

        lax.fori_loop(0, _NCH, body, 0)
        plsc.subcore_barrier()

        @pl.when(c == 0)
        def _():
            pltpu.sync_copy(acc.at[pl.ds(s * gs, gs)], cnt.at[pl.ds(s * gs, gs)])

        @pl.when(c == 1)
        def _():
            pltpu.sync_copy(acc.at[pl.ds(s * ds_, ds_)],
                            cnt.at[pl.ds(_N_GENE + s * ds_, ds_)])

    return k(dcat, ones_r, zer)


def _combine1_body(s_ref, c_ref, x_ref, wl_ref, wr_ref, b_ref, o_ref):
    inv = 1.0 / jnp.maximum(c_ref[:, 0:1], 1.0)
    mean = s_ref[...] * inv
    o_ref[...] = (
        jnp.dot(mean, wl_ref[...], preferred_element_type=jnp.float32)
        + jnp.dot(x_ref[...], wr_ref[...], preferred_element_type=jnp.float32)
        + b_ref[...]
    )


def _combine1(S, cnt, x_dst, Wl, Wr, b):
    n = S.shape[0]
    grid = (n // _ROW_BLK,)
    return pl.pallas_call(
        _combine1_body,
        grid=grid,
        in_specs=[
            pl.BlockSpec((_ROW_BLK, _D), lambda i: (i, 0)),
            pl.BlockSpec((_ROW_BLK, _HD), lambda i: (i, 0)),
            pl.BlockSpec((_ROW_BLK, _D), lambda i: (i, 0)),
            pl.BlockSpec((_D, _D), lambda i: (0, 0)),
            pl.BlockSpec((_D, _D), lambda i: (0, 0)),
            pl.BlockSpec((1, _D), lambda i: (0, 0)),
        ],
        out_specs=pl.BlockSpec((_ROW_BLK, _D), lambda i: (i, 0)),
        out_shape=jax.ShapeDtypeStruct((n, _D), jnp.float32),
    )(S, cnt, x_dst, Wl, Wr, b.reshape(1, _D))


def _combine2_head_body(s_ref, c_ref, x_ref, wl_ref, wr_ref, b_ref,
                        wmu_ref, bmu_ref, o_ref):
    inv = 1.0 / jnp.maximum(c_ref[:, 0:1], 1.0)
    mean = s_ref[...] * inv
    h = (
        jnp.dot(mean, wl_ref[...], preferred_element_type=jnp.float32)
        + jnp.dot(x_ref[...], wr_ref[...], preferred_element_type=jnp.float32)
        + b_ref[...]
    )
    o_ref[...] = jnp.dot(h, wmu_ref[...], preferred_element_type=jnp.float32) + bmu_ref[...]


def _combine2_head(S, cnt, x_dst, Wl, Wr, b, Wmu, bmu):
    n = S.shape[0]
    grid = (n // _ROW_BLK,)
    return pl.pallas_call(
        _combine2_head_body,
        grid=grid,
        in_specs=[
            pl.BlockSpec((_ROW_BLK, _D), lambda i: (i, 0)),
            pl.BlockSpec((_ROW_BLK, _HD), lambda i: (i, 0)),
            pl.BlockSpec((_ROW_BLK, _D), lambda i: (i, 0)),
            pl.BlockSpec((_D, _D), lambda i: (0, 0)),
            pl.BlockSpec((_D, _D), lambda i: (0, 0)),
            pl.BlockSpec((1, _D), lambda i: (0, 0)),
            pl.BlockSpec((_D, _OUT), lambda i: (0, 0)),
            pl.BlockSpec((1, _OUT), lambda i: (0, 0)),
        ],
        out_specs=pl.BlockSpec((_ROW_BLK, _OUT), lambda i: (i, 0)),
        out_shape=jax.ShapeDtypeStruct((n, _OUT), jnp.float32),
    )(S, cnt, x_dst, Wl, Wr, b.reshape(1, _D), Wmu, bmu.reshape(1, _OUT))


def _decoder_body(d_ref, g_ref, o_ref):
    o_ref[...] = jax.lax.dot_general(
        d_ref[...], g_ref[...], (((1,), (1,)), ((), ())),
        preferred_element_type=jnp.float32)


def _decoder(mu_d, mu_g):
    bm, bn = 512, 1024
    grid = (_N_DIS // bm, _N_GENE // bn)
    return pl.pallas_call(
        _decoder_body,
        grid=grid,
        in_specs=[
            pl.BlockSpec((bm, _OUT), lambda i, j: (i, 0)),
            pl.BlockSpec((bn, _OUT), lambda i, j: (j, 0)),
        ],
        out_specs=pl.BlockSpec((bm, bn), lambda i, j: (i, j)),
        out_shape=jax.ShapeDtypeStruct((_N_DIS, _N_GENE), jnp.float32),
    )(mu_d, mu_g)


def _cat_halves(x):
    return jnp.concatenate([x[:, :_HD], x[:, _HD:]], axis=0)


def _uncat(xc):
    n = xc.shape[0] // 2
    return jnp.concatenate([xc[:n], xc[n:]], axis=1)


def kernel(x_disease, x_gene, edge_index_d2g, edge_index_g2d,
           W_l1_d2g, W_r1_d2g, b1_d2g, W_l1_g2d, W_r1_g2d, b1_g2d,
           W_l2_d2g, W_r2_d2g, b2_d2g, W_l2_g2d, W_r2_g2d, b2_g2d,
           W_mu_d, b_mu_d, W_lv_d, b_lv_d, W_mu_g, b_mu_g, W_lv_g, b_lv_g):
    s1, d1i = edge_index_d2g[0], edge_index_d2g[1]
    s2, d2i = edge_index_g2d[0], edge_index_g2d[1]
    s1o = jnp.concatenate([s1, s1 + _N_DIS])
    s2o = jnp.concatenate([s2, s2 + _N_GENE])
    dcat = jnp.concatenate([d1i, d2i])
    zer = jnp.zeros((_N_GENE // _NS, _HD), jnp.float32)
    ones_r = jnp.ones((_C, _HD), jnp.float32)

    cnt = _sc_counts(dcat, ones_r, zer)
    cnt_g, cnt_d = cnt[:_N_GENE], cnt[_N_GENE:]

    Sg1, Sd1 = _sc_layer(_cat_halves(x_disease), _cat_halves(x_gene),
                         s1o, d1i, s2o, d2i, zer)
    g1 = _combine1(_uncat(Sg1), cnt_g, x_gene, W_l1_d2g, W_r1_d2g, b1_d2g)
    d1 = _combine1(_uncat(Sd1), cnt_d, x_disease, W_l1_g2d, W_r1_g2d, b1_g2d)

    Sg2, Sd2 = _sc_layer(_cat_halves(d1), _cat_halves(g1),
                         s1o, d1i, s2o, d2i, zer)
    mu_g = _combine2_head(_uncat(Sg2), cnt_g, g1,
                          W_l2_d2g, W_r2_d2g, b2_d2g, W_mu_g, b_mu_g)
    mu_d = _combine2_head(_uncat(Sd2), cnt_d, d1,
                          W_l2_g2d, W_r2_g2d, b2_g2d, W_mu_d, b_mu_d)

    return _decoder(mu_d, mu_g)


# speedup vs baseline: 7.4412x; 1.9389x over previous
import functools

import jax
import jax.numpy as jnp
from jax import lax
from jax.experimental import pallas as pl
from jax.experimental.pallas import tpu as pltpu
from jax.experimental.pallas import tpu_sc as plsc

_N_DIS = 4096
_N_GENE = 8192
_E = 262144
_D = 256
_HD = 128
_OUT = 64
_ROW_BLK = 256
_C = 128
_NS = 16
_EPS = _E // _NS
_NCH = _EPS // _C


def _mesh():
    return plsc.VectorSubcoreMesh(core_axis_name="c", subcore_axis_name="s")


def _sc_layer(xA_cat, xB_cat, s1o, d1i, s2o, d2i, zer):
    gs = _N_GENE // _NS
    ds_ = _N_DIS // _NS
    nrow = _EPS // _C
    prow = nrow // 2

    @functools.partial(
        pl.kernel,
        out_type=[
            jax.ShapeDtypeStruct((2 * _N_GENE, _HD), jnp.float32),
            jax.ShapeDtypeStruct((2 * _N_DIS, _HD), jnp.float32),
        ],
        mesh=_mesh(),
        scratch_types=[
            pltpu.VMEM((prow, _C), jnp.int32),
            pltpu.VMEM((prow, _C), jnp.int32),
            pltpu.VMEM((_C, _HD), jnp.float32),
            pltpu.VMEM((_C, _HD), jnp.float32),
            pltpu.VMEM_SHARED((_N_GENE, _HD), jnp.float32),
            pltpu.SemaphoreType.DMA,
            pltpu.SemaphoreType.DMA,
            pltpu.SemaphoreType.DMA,
            pltpu.SemaphoreType.DMA,
        ],
    )
    def k(xA, xB, s1r, d1r, s2r, d2r, zr, sg, sd,
          idx_s, idx_d, gbuf0, gbuf1, acc,
          gsem0, gsem1, ssem0, ssem1):
        c = lax.axis_index("c")
        s = lax.axis_index("s")

        def run_pass(table, srcr, dstr):
            for phase in range(2):
                pltpu.sync_copy(
                    srcr.at[pl.ds(c * (_E // _C) + s * nrow + phase * prow, prow)],
                    idx_s)
                pltpu.sync_copy(dstr.at[pl.ds(s * nrow + phase * prow, prow)],
                                idx_d)
                pltpu.async_copy(table.at[idx_s.at[0]], gbuf0, gsem0)
                pltpu.async_copy(table.at[idx_s.at[1]], gbuf1, gsem1)

                def body(j, carry):
                    a = 2 * j
                    b = 2 * j + 1
                    pltpu.make_async_copy(table.at[idx_s.at[a]], gbuf0, gsem0).wait()
                    pltpu.async_copy(gbuf0, acc.at[idx_d.at[a]], ssem0, add=True)
                    pltpu.make_async_copy(gbuf0, acc.at[idx_d.at[a]], ssem0).wait()

                    @pl.when(j < prow // 2 - 1)
                    def _():
                        pltpu.async_copy(table.at[idx_s.at[a + 2]], gbuf0, gsem0)

                    pltpu.make_async_copy(table.at[idx_s.at[b]], gbuf1, gsem1).wait()
                    pltpu.async_copy(gbuf1, acc.at[idx_d.at[b]], ssem1, add=True)
                    pltpu.make_async_copy(gbuf1, acc.at[idx_d.at[b]], ssem1).wait()

                    @pl.when(j < prow // 2 - 1)
                    def _():
                        pltpu.async_copy(table.at[idx_s.at[b + 2]], gbuf1, gsem1)

                    return carry

                lax.fori_loop(0, prow // 2, body, 0)

        pltpu.sync_copy(zr, acc.at[pl.ds(s * gs, gs)])
        plsc.subcore_barrier()
        run_pass(xA, s1r, d1r)
        plsc.subcore_barrier()
        pltpu.sync_copy(acc.at[pl.ds(s * gs, gs)],
                        sg.at[pl.ds(c * _N_GENE + s * gs, gs)])
        plsc.subcore_barrier()
        pltpu.sync_copy(zr.at[pl.ds(0, ds_)], acc.at[pl.ds(s * ds_, ds_)])
        plsc.subcore_barrier()
        run_pass(xB, s2r, d2r)
        plsc.subcore_barrier()
        pltpu.sync_copy(acc.at[pl.ds(s * ds_, ds_)],
                        sd.at[pl.ds(c * _N_DIS + s * ds_, ds_)])

    return k(xA_cat, xB_cat, s1o, d1i, s2o, d2i, zer)


def _sc_counts(dcat, ones_r, zer):
    gs = _N_GENE // _NS
    ds_ = _N_DIS // _NS

    @functools.partial(
        pl.kernel,
        out_type=jax.ShapeDtypeStruct((_N_GENE + _N_DIS, _HD), jnp.float32),
        mesh=_mesh(),
        scratch_types=[
            pltpu.VMEM((_EPS // _C, _C), jnp.int32),
            pltpu.VMEM((_C, _HD), jnp.float32),
            pltpu.VMEM_SHARED((_N_GENE, _HD), jnp.float32),
            pltpu.SemaphoreType.DMA,
        ],
    )
    def k(dr, onesr, zr, cnt, idx_d, obuf, acc, ssem):
        c = lax.axis_index("c")
        s = lax.axis_index("s")
        nrow = _EPS // _C
        pltpu.sync_copy(zr, acc.at[pl.ds(s * gs, gs)])
        pltpu.sync_copy(onesr, obuf)
        pltpu.sync_copy(dr.at[pl.ds(c * (_E // _C) + s * nrow, nrow)], idx_d)
        plsc.subcore_barrier()

        def body(i, carry):
            pltpu.async_copy(obuf, acc.at[idx_d.at[i]], ssem, add=True)
            return carry

        lax.fori_loop(0, nrow, body, 0)

        def drain(i, carry):
            pltpu.make_async_copy(obuf, acc.at[idx_d.at[0]], ssem).wait()
            return carry

        lax.fori_loop(0, nrow, drain, 0)
        plsc.subcore_barrier()

        @pl.when(c == 0)
        def _():
            pltpu.sync_copy(acc.at[pl.ds(s * gs, gs)], cnt.at[pl.ds(s * gs, gs)])

        @pl.when(c == 1)
        def _():
            pltpu.sync_copy(acc.at[pl.ds(s * ds_, ds_)],
                            cnt.at[pl.ds(_N_GENE + s * ds_, ds_)])

    return k(dcat, ones_r, zer)


def _combine1_body(s_ref, c_ref, x_ref, wl_ref, wr_ref, b_ref, o_ref):
    inv = 1.0 / jnp.maximum(c_ref[:, 0:1], 1.0)
    mean = s_ref[...] * inv
    o_ref[...] = (
        jnp.dot(mean, wl_ref[...], preferred_element_type=jnp.float32)
        + jnp.dot(x_ref[...], wr_ref[...], preferred_element_type=jnp.float32)
        + b_ref[...]
    )


def _combine1(S, cnt, x_dst, Wl, Wr, b):
    n = S.shape[0]
    grid = (n // _ROW_BLK,)
    return pl.pallas_call(
        _combine1_body,
        grid=grid,
        in_specs=[
            pl.BlockSpec((_ROW_BLK, _D), lambda i: (i, 0)),
            pl.BlockSpec((_ROW_BLK, _HD), lambda i: (i, 0)),
            pl.BlockSpec((_ROW_BLK, _D), lambda i: (i, 0)),
            pl.BlockSpec((_D, _D), lambda i: (0, 0)),
            pl.BlockSpec((_D, _D), lambda i: (0, 0)),
            pl.BlockSpec((1, _D), lambda i: (0, 0)),
        ],
        out_specs=pl.BlockSpec((_ROW_BLK, _D), lambda i: (i, 0)),
        out_shape=jax.ShapeDtypeStruct((n, _D), jnp.float32),
    )(S, cnt, x_dst, Wl, Wr, b.reshape(1, _D))


def _combine2_head_body(s_ref, c_ref, x_ref, wl_ref, wr_ref, b_ref,
                        wmu_ref, bmu_ref, o_ref):
    inv = 1.0 / jnp.maximum(c_ref[:, 0:1], 1.0)
    mean = s_ref[...] * inv
    h = (
        jnp.dot(mean, wl_ref[...], preferred_element_type=jnp.float32)
        + jnp.dot(x_ref[...], wr_ref[...], preferred_element_type=jnp.float32)
        + b_ref[...]
    )
    o_ref[...] = jnp.dot(h, wmu_ref[...], preferred_element_type=jnp.float32) + bmu_ref[...]


def _combine2_head(S, cnt, x_dst, Wl, Wr, b, Wmu, bmu):
    n = S.shape[0]
    grid = (n // _ROW_BLK,)
    return pl.pallas_call(
        _combine2_head_body,
        grid=grid,
        in_specs=[
            pl.BlockSpec((_ROW_BLK, _D), lambda i: (i, 0)),
            pl.BlockSpec((_ROW_BLK, _HD), lambda i: (i, 0)),
            pl.BlockSpec((_ROW_BLK, _D), lambda i: (i, 0)),
            pl.BlockSpec((_D, _D), lambda i: (0, 0)),
            pl.BlockSpec((_D, _D), lambda i: (0, 0)),
            pl.BlockSpec((1, _D), lambda i: (0, 0)),
            pl.BlockSpec((_D, _OUT), lambda i: (0, 0)),
            pl.BlockSpec((1, _OUT), lambda i: (0, 0)),
        ],
        out_specs=pl.BlockSpec((_ROW_BLK, _OUT), lambda i: (i, 0)),
        out_shape=jax.ShapeDtypeStruct((n, _OUT), jnp.float32),
    )(S, cnt, x_dst, Wl, Wr, b.reshape(1, _D), Wmu, bmu.reshape(1, _OUT))


def _decoder_body(d_ref, g_ref, o_ref):
    o_ref[...] = jax.lax.dot_general(
        d_ref[...], g_ref[...], (((1,), (1,)), ((), ())),
        preferred_element_type=jnp.float32)


def _decoder(mu_d, mu_g):
    bm, bn = 512, 1024
    grid = (_N_DIS // bm, _N_GENE // bn)
    return pl.pallas_call(
        _decoder_body,
        grid=grid,
        in_specs=[
            pl.BlockSpec((bm, _OUT), lambda i, j: (i, 0)),
            pl.BlockSpec((bn, _OUT), lambda i, j: (j, 0)),
        ],
        out_specs=pl.BlockSpec((bm, bn), lambda i, j: (i, j)),
        out_shape=jax.ShapeDtypeStruct((_N_DIS, _N_GENE), jnp.float32),
    )(mu_d, mu_g)


def _cat_halves(x):
    return jnp.concatenate([x[:, :_HD], x[:, _HD:]], axis=0)


def _uncat(xc):
    n = xc.shape[0] // 2
    return jnp.concatenate([xc[:n], xc[n:]], axis=1)


def kernel(x_disease, x_gene, edge_index_d2g, edge_index_g2d,
           W_l1_d2g, W_r1_d2g, b1_d2g, W_l1_g2d, W_r1_g2d, b1_g2d,
           W_l2_d2g, W_r2_d2g, b2_d2g, W_l2_g2d, W_r2_g2d, b2_g2d,
           W_mu_d, b_mu_d, W_lv_d, b_lv_d, W_mu_g, b_mu_g, W_lv_g, b_lv_g):
    s1, d1i = edge_index_d2g[0], edge_index_d2g[1]
    s2, d2i = edge_index_g2d[0], edge_index_g2d[1]
    s1o = jnp.concatenate([s1, s1 + _N_DIS]).reshape(2 * _E // _C, _C)
    s2o = jnp.concatenate([s2, s2 + _N_GENE]).reshape(2 * _E // _C, _C)
    dcat = jnp.concatenate([d1i, d2i]).reshape(2 * _E // _C, _C)
    d1r = d1i.reshape(_E // _C, _C)
    d2r = d2i.reshape(_E // _C, _C)
    zer = jnp.zeros((_N_GENE // _NS, _HD), jnp.float32)
    ones_r = jnp.ones((_C, _HD), jnp.float32)

    cnt = _sc_counts(dcat, ones_r, zer)
    cnt_g, cnt_d = cnt[:_N_GENE], cnt[_N_GENE:]

    Sg1, Sd1 = _sc_layer(_cat_halves(x_disease), _cat_halves(x_gene),
                         s1o, d1r, s2o, d2r, zer)
    g1 = _combine1(_uncat(Sg1), cnt_g, x_gene, W_l1_d2g, W_r1_d2g, b1_d2g)
    d1 = _combine1(_uncat(Sd1), cnt_d, x_disease, W_l1_g2d, W_r1_g2d, b1_g2d)

    Sg2, Sd2 = _sc_layer(_cat_halves(d1), _cat_halves(g1),
                         s1o, d1r, s2o, d2r, zer)
    mu_g = _combine2_head(_uncat(Sg2), cnt_g, g1,
                          W_l2_d2g, W_r2_d2g, b2_d2g, W_mu_g, b_mu_g)
    mu_d = _combine2_head(_uncat(Sd2), cnt_d, d1,
                          W_l2_g2d, W_r2_g2d, b2_g2d, W_mu_d, b_mu_d)

    return _decoder(mu_d, mu_g)


# speedup vs baseline: 12.6365x; 1.6982x over previous
import functools

import jax
import jax.numpy as jnp
from jax import lax
from jax.experimental import pallas as pl
from jax.experimental.pallas import tpu as pltpu
from jax.experimental.pallas import tpu_sc as plsc

_N_DIS = 4096
_N_GENE = 8192
_N_CAT = _N_DIS + _N_GENE
_E = 262144
_D = 256
_HD = 128
_OUT = 64
_ROW_BLK = 256
_C = 128
_NS = 16
_EPS = _E // _NS
_NROW = _EPS // _C
_PROW = _NROW // 2
_GS = _N_GENE // _NS
_DS = _N_DIS // _NS


def _mesh():
    return plsc.VectorSubcoreMesh(core_axis_name="c", subcore_axis_name="s")


def _sc_seg(tcat, scat, dcat, zer):
    @functools.partial(
        pl.kernel,
        out_type=jax.ShapeDtypeStruct((_N_CAT, _HD), jnp.float32),
        mesh=_mesh(),
        scratch_types=[
            pltpu.VMEM((_PROW, _C), jnp.int32),
            pltpu.VMEM((_PROW, _C), jnp.int32),
            pltpu.VMEM((_C, _HD), jnp.float32),
            pltpu.VMEM((_C, _HD), jnp.float32),
            pltpu.VMEM_SHARED((_N_GENE, _HD), jnp.float32),
            pltpu.SemaphoreType.DMA,
            pltpu.SemaphoreType.DMA,
            pltpu.SemaphoreType.DMA,
            pltpu.SemaphoreType.DMA,
        ],
    )
    def k(tr, sr, dr, zr, out,
          idx_s, idx_d, gbuf0, gbuf1, acc,
          gsem0, gsem1, ssem0, ssem1):
        c = lax.axis_index("c")
        s = lax.axis_index("s")

        pltpu.sync_copy(zr, acc.at[pl.ds(s * _GS, _GS)])
        plsc.subcore_barrier()

        for phase in range(2):
            off = c * (_E // _C) + s * _NROW + phase * _PROW
            pltpu.sync_copy(sr.at[pl.ds(off, _PROW)], idx_s)
            pltpu.sync_copy(dr.at[pl.ds(off, _PROW)], idx_d)
            pltpu.async_copy(tr.at[idx_s.at[0]], gbuf0, gsem0)
            pltpu.async_copy(tr.at[idx_s.at[1]], gbuf1, gsem1)

            def body(j, carry):
                a = 2 * j
                b = 2 * j + 1
                pltpu.make_async_copy(tr.at[idx_s.at[a]], gbuf0, gsem0).wait()
                pltpu.async_copy(gbuf0, acc.at[idx_d.at[a]], ssem0, add=True)
                pltpu.make_async_copy(gbuf0, acc.at[idx_d.at[a]], ssem0).wait()

                @pl.when(j < _PROW // 2 - 1)
                def _():
                    pltpu.async_copy(tr.at[idx_s.at[a + 2]], gbuf0, gsem0)

                pltpu.make_async_copy(tr.at[idx_s.at[b]], gbuf1, gsem1).wait()
                pltpu.async_copy(gbuf1, acc.at[idx_d.at[b]], ssem1, add=True)
                pltpu.make_async_copy(gbuf1, acc.at[idx_d.at[b]], ssem1).wait()

                @pl.when(j < _PROW // 2 - 1)
                def _():
                    pltpu.async_copy(tr.at[idx_s.at[b + 2]], gbuf1, gsem1)

                return carry

            lax.fori_loop(0, _PROW // 2, body, 0)

        plsc.subcore_barrier()

        @pl.when(c == 0)
        def _():
            pltpu.sync_copy(acc.at[pl.ds(s * _GS, _GS)],
                            out.at[pl.ds(_N_DIS + s * _GS, _GS)])

        @pl.when(c == 1)
        def _():
            pltpu.sync_copy(acc.at[pl.ds(s * _DS, _DS)],
                            out.at[pl.ds(s * _DS, _DS)])

    return k(tcat, scat, dcat, zer)


def _sc_counts(dcat, ones_r, zer):
    @functools.partial(
        pl.kernel,
        out_type=jax.ShapeDtypeStruct((_N_CAT, _HD), jnp.float32),
        mesh=_mesh(),
        scratch_types=[
            pltpu.VMEM((_NROW, _C), jnp.int32),
            pltpu.VMEM((_C, _HD), jnp.float32),
            pltpu.VMEM_SHARED((_N_GENE, _HD), jnp.float32),
            pltpu.SemaphoreType.DMA,
        ],
    )
    def k(dr, onesr, zr, cnt, idx_d, obuf, acc, ssem):
        c = lax.axis_index("c")
        s = lax.axis_index("s")
        pltpu.sync_copy(zr, acc.at[pl.ds(s * _GS, _GS)])
        pltpu.sync_copy(onesr, obuf)
        pltpu.sync_copy(dr.at[pl.ds(c * (_E // _C) + s * _NROW, _NROW)], idx_d)
        plsc.subcore_barrier()

        def body(i, carry):
            pltpu.async_copy(obuf, acc.at[idx_d.at[i]], ssem, add=True)
            return carry

        lax.fori_loop(0, _NROW, body, 0)

        def drain(i, carry):
            pltpu.make_async_copy(obuf, acc.at[idx_d.at[0]], ssem).wait()
            return carry

        lax.fori_loop(0, _NROW, drain, 0)
        plsc.subcore_barrier()

        @pl.when(c == 0)
        def _():
            pltpu.sync_copy(acc.at[pl.ds(s * _GS, _GS)],
                            cnt.at[pl.ds(_N_DIS + s * _GS, _GS)])

        @pl.when(c == 1)
        def _():
            pltpu.sync_copy(acc.at[pl.ds(s * _DS, _DS)],
                            cnt.at[pl.ds(s * _DS, _DS)])

    return k(dcat, ones_r, zer)


def _compose_body(wl1dg_ref, wr1dg_ref, b1dg_ref, wl1gd_ref, wr1gd_ref,
                  b1gd_ref, wl2dg_ref, wr2dg_ref, b2dg_ref, wl2gd_ref,
                  wr2gd_ref, b2gd_ref, wmud_ref, bmud_ref, wmug_ref,
                  bmug_ref, wqd_ref, wxg_ref, bxg_ref, wqg_ref, wxd_ref,
                  bxd_ref):
    dot = functools.partial(jnp.dot, preferred_element_type=jnp.float32)
    a_d = dot(wl2gd_ref[...], wmud_ref[...])
    b_g = dot(wr2dg_ref[...], wmug_ref[...])
    c_g = dot(b2dg_ref[...], wmug_ref[...]) + bmug_ref[...]
    a_g = dot(wl2dg_ref[...], wmug_ref[...])
    b_d = dot(wr2gd_ref[...], wmud_ref[...])
    c_d = dot(b2gd_ref[...], wmud_ref[...]) + bmud_ref[...]

    wqd_ref[:, :_OUT] = dot(wl1dg_ref[...], a_d)
    wqd_ref[:, _OUT:] = dot(wl1dg_ref[...], b_g)
    wxg_ref[:, :_OUT] = dot(wr1dg_ref[...], a_d)
    wxg_ref[:, _OUT:] = dot(wr1dg_ref[...], b_g)
    bxg_ref[:, :_OUT] = dot(b1dg_ref[...], a_d)
    bxg_ref[:, _OUT:] = dot(b1dg_ref[...], b_g) + c_g

    wqg_ref[:, :_OUT] = dot(wl1gd_ref[...], a_g)
    wqg_ref[:, _OUT:] = dot(wl1gd_ref[...], b_d)
    wxd_ref[:, :_OUT] = dot(wr1gd_ref[...], a_g)
    wxd_ref[:, _OUT:] = dot(wr1gd_ref[...], b_d)
    bxd_ref[:, :_OUT] = dot(b1gd_ref[...], a_g)
    bxd_ref[:, _OUT:] = dot(b1gd_ref[...], b_d) + c_d


def _compose(wl1dg, wr1dg, b1dg, wl1gd, wr1gd, b1gd,
             wl2dg, wr2dg, b2dg, wl2gd, wr2gd, b2gd,
             wmud, bmud, wmug, bmug):
    full = lambda shp: pl.BlockSpec(shp, lambda: tuple(0 for _ in shp))
    mat = full((_D, _D))
    row = full((1, _D))
    omat = full((_D, _HD))
    orow = full((1, _HD))
    return pl.pallas_call(
        _compose_body,
        grid=(),
        in_specs=[mat, mat, row, mat, mat, row,
                  mat, mat, row, mat, mat, row,
                  full((_D, _OUT)), full((1, _OUT)),
                  full((_D, _OUT)), full((1, _OUT))],
        out_specs=[omat, omat, orow, omat, omat, orow],
        out_shape=[
            jax.ShapeDtypeStruct((_D, _HD), jnp.float32),
            jax.ShapeDtypeStruct((_D, _HD), jnp.float32),
            jax.ShapeDtypeStruct((1, _HD), jnp.float32),
            jax.ShapeDtypeStruct((_D, _HD), jnp.float32),
            jax.ShapeDtypeStruct((_D, _HD), jnp.float32),
            jax.ShapeDtypeStruct((1, _HD), jnp.float32),
        ],
    )(wl1dg, wr1dg, b1dg.reshape(1, _D), wl1gd, wr1gd, b1gd.reshape(1, _D),
      wl2dg, wr2dg, b2dg.reshape(1, _D), wl2gd, wr2gd, b2gd.reshape(1, _D),
      wmud, bmud.reshape(1, _OUT), wmug, bmug.reshape(1, _OUT))


def _proj_body(x_ref, wq_ref, wx_ref, bx_ref, q_ref, x_out_ref):
    q_ref[...] = jnp.dot(x_ref[...], wq_ref[...],
                         preferred_element_type=jnp.float32)
    x_out_ref[...] = (
        jnp.dot(x_ref[...], wx_ref[...], preferred_element_type=jnp.float32)
        + bx_ref[...]
    )


def _proj(x, wq, wx, bx):
    n = x.shape[0]
    grid = (n // _ROW_BLK,)
    blk = pl.BlockSpec((_ROW_BLK, _D), lambda i: (i, 0))
    oblk = pl.BlockSpec((_ROW_BLK, _HD), lambda i: (i, 0))
    return pl.pallas_call(
        _proj_body,
        grid=grid,
        in_specs=[blk,
                  pl.BlockSpec((_D, _HD), lambda i: (0, 0)),
                  pl.BlockSpec((_D, _HD), lambda i: (0, 0)),
                  pl.BlockSpec((1, _HD), lambda i: (0, 0))],
        out_specs=[oblk, oblk],
        out_shape=[jax.ShapeDtypeStruct((n, _HD), jnp.float32),
                   jax.ShapeDtypeStruct((n, _HD), jnp.float32)],
    )(x, wq, wx, bx)


def _tu_body(r_ref, c_ref, x_ref, o_ref):
    inv = 1.0 / jnp.maximum(c_ref[...], 1.0)
    o_ref[...] = r_ref[...] * inv + x_ref[...]


def _tu(r1, cnt, xcat):
    grid = (_N_CAT // _ROW_BLK,)
    blk = pl.BlockSpec((_ROW_BLK, _HD), lambda i: (i, 0))
    return pl.pallas_call(
        _tu_body,
        grid=grid,
        in_specs=[blk, blk, blk],
        out_specs=blk,
        out_shape=jax.ShapeDtypeStruct((_N_CAT, _HD), jnp.float32),
    )(r1, cnt, xcat)


def _decoder_body(sd_ref, cd_ref, tud_ref, sg_ref, cg_ref, tug_ref, o_ref):
    inv_d = 1.0 / jnp.maximum(cd_ref[:, :_OUT], 1.0)
    mu_d = sd_ref[:, :_OUT] * inv_d + tud_ref[:, _OUT:]
    inv_g = 1.0 / jnp.maximum(cg_ref[:, :_OUT], 1.0)
    mu_g = sg_ref[:, :_OUT] * inv_g + tug_ref[:, _OUT:]
    o_ref[...] = jax.lax.dot_general(
        mu_d, mu_g, (((1,), (1,)), ((), ())),
        preferred_element_type=jnp.float32)


def _decoder(s2, cnt, tu):
    bm, bn = 512, 1024
    grid = (_N_DIS // bm, _N_GENE // bn)
    dblk = pl.BlockSpec((bm, _HD), lambda i, j: (i, 0))
    gblk = pl.BlockSpec((bn, _HD), lambda i, j: (_N_DIS // bn + j, 0))
    return pl.pallas_call(
        _decoder_body,
        grid=grid,
        in_specs=[dblk, dblk, dblk, gblk, gblk, gblk],
        out_specs=pl.BlockSpec((bm, bn), lambda i, j: (i, j)),
        out_shape=jax.ShapeDtypeStruct((_N_DIS, _N_GENE), jnp.float32),
    )(s2, cnt, tu, s2, cnt, tu)


def kernel(x_disease, x_gene, edge_index_d2g, edge_index_g2d,
           W_l1_d2g, W_r1_d2g, b1_d2g, W_l1_g2d, W_r1_g2d, b1_g2d,
           W_l2_d2g, W_r2_d2g, b2_d2g, W_l2_g2d, W_r2_g2d, b2_g2d,
           W_mu_d, b_mu_d, W_lv_d, b_lv_d, W_mu_g, b_mu_g, W_lv_g, b_lv_g):
    s1, d1i = edge_index_d2g[0], edge_index_d2g[1]
    s2, d2i = edge_index_g2d[0], edge_index_g2d[1]
    scat = jnp.concatenate([s1, s2 + _N_DIS]).reshape(2 * _E // _C, _C)
    dcat = jnp.concatenate([d1i, d2i]).reshape(2 * _E // _C, _C)
    zer = jnp.zeros((_GS, _HD), jnp.float32)
    ones_r = jnp.ones((_C, _HD), jnp.float32)

    wqd, wxg, bxg, wqg, wxd, bxd = _compose(
        W_l1_d2g, W_r1_d2g, b1_d2g, W_l1_g2d, W_r1_g2d, b1_g2d,
        W_l2_d2g, W_r2_d2g, b2_d2g, W_l2_g2d, W_r2_g2d, b2_g2d,
        W_mu_d, b_mu_d, W_mu_g, b_mu_g)

    q_d, x_d = _proj(x_disease, wqd, wxd, bxd)
    q_g, x_g = _proj(x_gene, wqg, wxg, bxg)
    qcat = jnp.concatenate([q_d, q_g], axis=0)
    xcat = jnp.concatenate([x_d, x_g], axis=0)

    cnt = _sc_counts(dcat, ones_r, zer)
    r1 = _sc_seg(qcat, scat, dcat, zer)
    tu = _tu(r1, cnt, xcat)
    s2sum = _sc_seg(tu, scat, dcat, zer)
    return _decoder(s2sum, cnt, tu)


# speedup vs baseline: 12.6872x; 1.0040x over previous
import functools

import jax
import jax.numpy as jnp
from jax import lax
from jax.experimental import pallas as pl
from jax.experimental.pallas import tpu as pltpu
from jax.experimental.pallas import tpu_sc as plsc

_N_DIS = 4096
_N_GENE = 8192
_N_CAT = _N_DIS + _N_GENE
_E = 262144
_D = 256
_HD = 128
_OUT = 64
_ROW_BLK = 256
_C = 128
_NS = 16
_EPS = _E // _NS
_NROW = _EPS // _C
_NPH = 4
_PROW = _NROW // _NPH
_GS = _N_GENE // _NS
_DS = _N_DIS // _NS


def _mesh():
    return plsc.VectorSubcoreMesh(core_axis_name="c", subcore_axis_name="s")


def _sc_seg(tcat, scat, dcat, zer):
    @functools.partial(
        pl.kernel,
        out_type=jax.ShapeDtypeStruct((_N_CAT, _HD), jnp.float32),
        mesh=_mesh(),
        scratch_types=[
            pltpu.VMEM((_PROW, _C), jnp.int32),
            pltpu.VMEM((_PROW, _C), jnp.int32),
            pltpu.VMEM((_C, _HD), jnp.float32),
            pltpu.VMEM((_C, _HD), jnp.float32),
            pltpu.VMEM((_C, _HD), jnp.float32),
            pltpu.VMEM_SHARED((_N_GENE, _HD), jnp.float32),
            pltpu.SemaphoreType.DMA,
            pltpu.SemaphoreType.DMA,
            pltpu.SemaphoreType.DMA,
            pltpu.SemaphoreType.DMA,
            pltpu.SemaphoreType.DMA,
            pltpu.SemaphoreType.DMA,
        ],
    )
    def k(tr, sr, dr, zr, out,
          idx_s, idx_d, gbuf0, gbuf1, gbuf2, acc,
          gsem0, gsem1, gsem2, ssem0, ssem1, ssem2):
        c = lax.axis_index("c")
        s = lax.axis_index("s")
        bufs = (gbuf0, gbuf1, gbuf2)
        gsems = (gsem0, gsem1, gsem2)
        ssems = (ssem0, ssem1, ssem2)

        pltpu.sync_copy(zr, acc.at[pl.ds(s * _GS, _GS)])
        plsc.subcore_barrier()

        def gwait(k_, row):
            pltpu.make_async_copy(tr.at[idx_s.at[row]], bufs[k_],
                                  gsems[k_]).wait()

        def sissue(k_, row):
            pltpu.async_copy(bufs[k_], acc.at[idx_d.at[row]], ssems[k_],
                             add=True)

        def swait(k_, row):
            pltpu.make_async_copy(bufs[k_], acc.at[idx_d.at[row]],
                                  ssems[k_]).wait()

        def gissue(k_, row):
            pltpu.async_copy(tr.at[idx_s.at[row]], bufs[k_], gsems[k_])

        for phase in range(_NPH):
            off = c * (_E // _C) + s * _NROW + phase * _PROW
            pltpu.sync_copy(sr.at[pl.ds(off, _PROW)], idx_s)
            pltpu.sync_copy(dr.at[pl.ds(off, _PROW)], idx_d)
            gissue(0, 0)
            gissue(1, 1)
            gwait(0, 0); sissue(0, 0); gissue(2, 2)
            gwait(1, 1); sissue(1, 1); swait(0, 0); gissue(0, 3)
            gwait(2, 2); sissue(2, 2); swait(1, 1); gissue(1, 4)

            def body(i, carry):
                base = 3 + 3 * i
                for k_ in range(3):
                    a = base + k_
                    m = (k_ + 2) % 3
                    gwait(k_, a)
                    sissue(k_, a)
                    swait(m, a - 1)
                    gissue(m, a + 2)
                return carry

            lax.fori_loop(0, (_PROW - 5) // 3, body, 0)
            gwait(0, _PROW - 2); sissue(0, _PROW - 2); swait(2, _PROW - 3)
            gwait(1, _PROW - 1); sissue(1, _PROW - 1); swait(0, _PROW - 2)
            swait(1, _PROW - 1)

        plsc.subcore_barrier()

        @pl.when(c == 0)
        def _():
            pltpu.sync_copy(acc.at[pl.ds(s * _GS, _GS)],
                            out.at[pl.ds(_N_DIS + s * _GS, _GS)])

        @pl.when(c == 1)
        def _():
            pltpu.sync_copy(acc.at[pl.ds(s * _DS, _DS)],
                            out.at[pl.ds(s * _DS, _DS)])

    return k(tcat, scat, dcat, zer)


def _sc_counts(dcat, ones_r, zer):
    @functools.partial(
        pl.kernel,
        out_type=jax.ShapeDtypeStruct((_N_CAT, _HD), jnp.float32),
        mesh=_mesh(),
        scratch_types=[
            pltpu.VMEM((_NROW, _C), jnp.int32),
            pltpu.VMEM((_C, _HD), jnp.float32),
            pltpu.VMEM_SHARED((_N_GENE, _HD), jnp.float32),
            pltpu.SemaphoreType.DMA,
        ],
    )
    def k(dr, onesr, zr, cnt, idx_d, obuf, acc, ssem):
        c = lax.axis_index("c")
        s = lax.axis_index("s")
        pltpu.sync_copy(zr, acc.at[pl.ds(s * _GS, _GS)])
        pltpu.sync_copy(onesr, obuf)
        pltpu.sync_copy(dr.at[pl.ds(c * (_E // _C) + s * _NROW, _NROW)], idx_d)
        plsc.subcore_barrier()

        def body(i, carry):
            pltpu.async_copy(obuf, acc.at[idx_d.at[i]], ssem, add=True)
            return carry

        lax.fori_loop(0, _NROW, body, 0)

        def drain(i, carry):
            pltpu.make_async_copy(obuf, acc.at[idx_d.at[0]], ssem).wait()
            return carry

        lax.fori_loop(0, _NROW, drain, 0)
        plsc.subcore_barrier()

        @pl.when(c == 0)
        def _():
            pltpu.sync_copy(acc.at[pl.ds(s * _GS, _GS)],
                            cnt.at[pl.ds(_N_DIS + s * _GS, _GS)])

        @pl.when(c == 1)
        def _():
            pltpu.sync_copy(acc.at[pl.ds(s * _DS, _DS)],
                            cnt.at[pl.ds(s * _DS, _DS)])

    return k(dcat, ones_r, zer)


def _compose_body(wl1dg_ref, wr1dg_ref, b1dg_ref, wl1gd_ref, wr1gd_ref,
                  b1gd_ref, wl2dg_ref, wr2dg_ref, b2dg_ref, wl2gd_ref,
                  wr2gd_ref, b2gd_ref, wmud_ref, bmud_ref, wmug_ref,
                  bmug_ref, wqd_ref, wxg_ref, bxg_ref, wqg_ref, wxd_ref,
                  bxd_ref):
    dot = functools.partial(jnp.dot, preferred_element_type=jnp.float32)
    a_d = dot(wl2gd_ref[...], wmud_ref[...])
    b_g = dot(wr2dg_ref[...], wmug_ref[...])
    c_g = dot(b2dg_ref[...], wmug_ref[...]) + bmug_ref[...]
    a_g = dot(wl2dg_ref[...], wmug_ref[...])
    b_d = dot(wr2gd_ref[...], wmud_ref[...])
    c_d = dot(b2gd_ref[...], wmud_ref[...]) + bmud_ref[...]

    wqd_ref[:, :_OUT] = dot(wl1dg_ref[...], a_d)
    wqd_ref[:, _OUT:] = dot(wl1dg_ref[...], b_g)
    wxg_ref[:, :_OUT] = dot(wr1dg_ref[...], a_d)
    wxg_ref[:, _OUT:] = dot(wr1dg_ref[...], b_g)
    bxg_ref[:, :_OUT] = dot(b1dg_ref[...], a_d)
    bxg_ref[:, _OUT:] = dot(b1dg_ref[...], b_g) + c_g

    wqg_ref[:, :_OUT] = dot(wl1gd_ref[...], a_g)
    wqg_ref[:, _OUT:] = dot(wl1gd_ref[...], b_d)
    wxd_ref[:, :_OUT] = dot(wr1gd_ref[...], a_g)
    wxd_ref[:, _OUT:] = dot(wr1gd_ref[...], b_d)
    bxd_ref[:, :_OUT] = dot(b1gd_ref[...], a_g)
    bxd_ref[:, _OUT:] = dot(b1gd_ref[...], b_d) + c_d


def _compose(wl1dg, wr1dg, b1dg, wl1gd, wr1gd, b1gd,
             wl2dg, wr2dg, b2dg, wl2gd, wr2gd, b2gd,
             wmud, bmud, wmug, bmug):
    full = lambda shp: pl.BlockSpec(shp, lambda: tuple(0 for _ in shp))
    mat = full((_D, _D))
    row = full((1, _D))
    omat = full((_D, _HD))
    orow = full((1, _HD))
    return pl.pallas_call(
        _compose_body,
        grid=(),
        in_specs=[mat, mat, row, mat, mat, row,
                  mat, mat, row, mat, mat, row,
                  full((_D, _OUT)), full((1, _OUT)),
                  full((_D, _OUT)), full((1, _OUT))],
        out_specs=[omat, omat, orow, omat, omat, orow],
        out_shape=[
            jax.ShapeDtypeStruct((_D, _HD), jnp.float32),
            jax.ShapeDtypeStruct((_D, _HD), jnp.float32),
            jax.ShapeDtypeStruct((1, _HD), jnp.float32),
            jax.ShapeDtypeStruct((_D, _HD), jnp.float32),
            jax.ShapeDtypeStruct((_D, _HD), jnp.float32),
            jax.ShapeDtypeStruct((1, _HD), jnp.float32),
        ],
    )(wl1dg, wr1dg, b1dg.reshape(1, _D), wl1gd, wr1gd, b1gd.reshape(1, _D),
      wl2dg, wr2dg, b2dg.reshape(1, _D), wl2gd, wr2gd, b2gd.reshape(1, _D),
      wmud, bmud.reshape(1, _OUT), wmug, bmug.reshape(1, _OUT))


def _proj_body(x_ref, wq_ref, wx_ref, bx_ref, q_ref, x_out_ref):
    q_ref[...] = jnp.dot(x_ref[...], wq_ref[...],
                         preferred_element_type=jnp.float32)
    x_out_ref[...] = (
        jnp.dot(x_ref[...], wx_ref[...], preferred_element_type=jnp.float32)
        + bx_ref[...]
    )


def _proj(x, wq, wx, bx):
    n = x.shape[0]
    grid = (n // _ROW_BLK,)
    blk = pl.BlockSpec((_ROW_BLK, _D), lambda i: (i, 0))
    oblk = pl.BlockSpec((_ROW_BLK, _HD), lambda i: (i, 0))
    return pl.pallas_call(
        _proj_body,
        grid=grid,
        in_specs=[blk,
                  pl.BlockSpec((_D, _HD), lambda i: (0, 0)),
                  pl.BlockSpec((_D, _HD), lambda i: (0, 0)),
                  pl.BlockSpec((1, _HD), lambda i: (0, 0))],
        out_specs=[oblk, oblk],
        out_shape=[jax.ShapeDtypeStruct((n, _HD), jnp.float32),
                   jax.ShapeDtypeStruct((n, _HD), jnp.float32)],
    )(x, wq, wx, bx)


def _tu_body(r_ref, c_ref, x_ref, o_ref):
    inv = 1.0 / jnp.maximum(c_ref[...], 1.0)
    o_ref[...] = r_ref[...] * inv + x_ref[...]


def _tu(r1, cnt, xcat):
    grid = (_N_CAT // _ROW_BLK,)
    blk = pl.BlockSpec((_ROW_BLK, _HD), lambda i: (i, 0))
    return pl.pallas_call(
        _tu_body,
        grid=grid,
        in_specs=[blk, blk, blk],
        out_specs=blk,
        out_shape=jax.ShapeDtypeStruct((_N_CAT, _HD), jnp.float32),
    )(r1, cnt, xcat)


def _decoder_body(sd_ref, cd_ref, tud_ref, sg_ref, cg_ref, tug_ref, o_ref):
    inv_d = 1.0 / jnp.maximum(cd_ref[:, :_OUT], 1.0)
    mu_d = sd_ref[:, :_OUT] * inv_d + tud_ref[:, _OUT:]
    inv_g = 1.0 / jnp.maximum(cg_ref[:, :_OUT], 1.0)
    mu_g = sg_ref[:, :_OUT] * inv_g + tug_ref[:, _OUT:]
    o_ref[...] = jax.lax.dot_general(
        mu_d, mu_g, (((1,), (1,)), ((), ())),
        preferred_element_type=jnp.float32)


def _decoder(s2, cnt, tu):
    bm, bn = 512, 1024
    grid = (_N_DIS // bm, _N_GENE // bn)
    dblk = pl.BlockSpec((bm, _HD), lambda i, j: (i, 0))
    gblk = pl.BlockSpec((bn, _HD), lambda i, j: (_N_DIS // bn + j, 0))
    return pl.pallas_call(
        _decoder_body,
        grid=grid,
        in_specs=[dblk, dblk, dblk, gblk, gblk, gblk],
        out_specs=pl.BlockSpec((bm, bn), lambda i, j: (i, j)),
        out_shape=jax.ShapeDtypeStruct((_N_DIS, _N_GENE), jnp.float32),
    )(s2, cnt, tu, s2, cnt, tu)


def kernel(x_disease, x_gene, edge_index_d2g, edge_index_g2d,
           W_l1_d2g, W_r1_d2g, b1_d2g, W_l1_g2d, W_r1_g2d, b1_g2d,
           W_l2_d2g, W_r2_d2g, b2_d2g, W_l2_g2d, W_r2_g2d, b2_g2d,
           W_mu_d, b_mu_d, W_lv_d, b_lv_d, W_mu_g, b_mu_g, W_lv_g, b_lv_g):
    s1, d1i = edge_index_d2g[0], edge_index_d2g[1]
    s2, d2i = edge_index_g2d[0], edge_index_g2d[1]
    scat = jnp.concatenate([s1, s2 + _N_DIS]).reshape(2 * _E // _C, _C)
    dcat = jnp.concatenate([d1i, d2i]).reshape(2 * _E // _C, _C)
    zer = jnp.zeros((_GS, _HD), jnp.float32)
    ones_r = jnp.ones((_C, _HD), jnp.float32)

    wqd, wxg, bxg, wqg, wxd, bxd = _compose(
        W_l1_d2g, W_r1_d2g, b1_d2g, W_l1_g2d, W_r1_g2d, b1_g2d,
        W_l2_d2g, W_r2_d2g, b2_d2g, W_l2_g2d, W_r2_g2d, b2_g2d,
        W_mu_d, b_mu_d, W_mu_g, b_mu_g)

    q_d, x_d = _proj(x_disease, wqd, wxd, bxd)
    q_g, x_g = _proj(x_gene, wqg, wxg, bxg)
    qcat = jnp.concatenate([q_d, q_g], axis=0)
    xcat = jnp.concatenate([x_d, x_g], axis=0)

    cnt = _sc_counts(dcat, ones_r, zer)
    r1 = _sc_seg(qcat, scat, dcat, zer)
    tu = _tu(r1, cnt, xcat)
    s2sum = _sc_seg(tu, scat, dcat, zer)
    return _decoder(s2sum, cnt, tu)
